# Initial kernel scaffold; baseline (speedup 1.0000x reference)
import jax
import jax.numpy as jnp
from jax.experimental import pallas as pl


def kernel(image):
    raise NotImplementedError("write your pallas kernel here")


# speedup vs baseline: 1.8135x; 1.8135x over previous
import functools

import jax
import jax.numpy as jnp
import numpy as np
from jax.experimental import pallas as pl
from jax.experimental.pallas import tpu as pltpu
from jax.experimental.pallas import tpu_sc as plsc

_X_TILES = 4
_Y_TILES = 4
_WINDOW = 128


@functools.lru_cache(maxsize=None)
def _src_row_table(B, C, H, W):
    hs, ws = _Y_TILES, _X_TILES
    h, w = H // hs, W // ws
    with jax.ensure_compile_time_eval():
        u = jax.random.uniform(jax.random.key(42), (B, hs * ws))
        perm = np.asarray(jnp.argsort(u, axis=-1))
    si, sj = perm // ws, perm % ws

    b = np.arange(B)[:, None, None, None, None]
    c = np.arange(C)[None, :, None, None, None]
    ti = np.arange(hs)[None, None, :, None, None]
    dy = np.arange(h)[None, None, None, :, None]
    tj = np.arange(ws)[None, None, None, None, :]
    t = ti * ws + tj
    s_i = si[b, t]
    s_j = sj[b, t]
    src = ((b * C + c) * H + s_i * h + dy) * ws + s_j
    return np.ascontiguousarray(src.reshape(1, -1).astype(np.int32))


def kernel(image):
    B, C, H, W = image.shape
    w = W // _X_TILES
    R = B * C * H * _X_TILES

    idx = jnp.asarray(_src_row_table(B, C, H, W))
    rows = image.reshape(R, w)

    mesh = plsc.VectorSubcoreMesh(core_axis_name="core", subcore_axis_name="subcore")

    @functools.partial(
        pl.kernel,
        out_type=jax.ShapeDtypeStruct((R, w), image.dtype),
        mesh=mesh,
        compiler_params=pltpu.CompilerParams(use_tc_tiling_on_sc=False),
    )
    def gather_rows(x_hbm, i_hbm, o_hbm):
        def body(i_vmem, o_vmem):
            pltpu.sync_copy(x_hbm.at[i_vmem.at[0]], o_vmem)

        pltpu.emit_pipeline(
            body,
            grid=(R // _WINDOW,),
            in_specs=[pl.BlockSpec((1, _WINDOW), lambda i: (0, i))],
            out_specs=[pl.BlockSpec((_WINDOW, w), lambda i: (i, 0))],
            core_axis_name=("core", "subcore"),
            dimension_semantics=(pltpu.PARALLEL,),
        )(i_hbm, o_hbm)

    out = gather_rows(rows, idx)
    return out.reshape(B, C, H, W)


# speedup vs baseline: 1.9941x; 1.0996x over previous
import functools

import jax
import jax.numpy as jnp
import numpy as np
from jax import lax
from jax.experimental import pallas as pl
from jax.experimental.pallas import tpu as pltpu
from jax.experimental.pallas import tpu_sc as plsc

_X_TILES = 4
_Y_TILES = 4

_NWORKERS = 32
_K = 128
_GC = 4
_GROUP_ROWS = _K * _GC


@functools.lru_cache(maxsize=None)
def _src_row_table(B, C, H, W):
    hs, ws = _Y_TILES, _X_TILES
    h, w = H // hs, W // ws
    with jax.ensure_compile_time_eval():
        u = jax.random.uniform(jax.random.key(42), (B, hs * ws))
        perm = np.asarray(jnp.argsort(u, axis=-1))
    si, sj = perm // ws, perm % ws

    b = np.arange(B)[:, None, None, None, None]
    c = np.arange(C)[None, :, None, None, None]
    ti = np.arange(hs)[None, None, :, None, None]
    dy = np.arange(h)[None, None, None, :, None]
    tj = np.arange(ws)[None, None, None, None, :]
    t = ti * ws + tj
    s_i = si[b, t]
    s_j = sj[b, t]
    src = ((b * C + c) * H + s_i * h + dy) * ws + s_j
    return np.ascontiguousarray(src.reshape(-1).astype(np.int32))


def kernel(image):
    B, C, H, W = image.shape
    w = W // _X_TILES
    R = B * C * H * _X_TILES
    rows_per_worker = R // _NWORKERS
    nchunk = rows_per_worker // _K
    ngroup = rows_per_worker // _GROUP_ROWS

    idx = jnp.asarray(
        _src_row_table(B, C, H, W).reshape(_NWORKERS, nchunk, _K)
    )
    rows = image.reshape(R, w)

    mesh = plsc.VectorSubcoreMesh(core_axis_name="core", subcore_axis_name="subcore")

    @functools.partial(
        pl.kernel,
        out_type=jax.ShapeDtypeStruct((R, w), image.dtype),
        mesh=mesh,
        compiler_params=pltpu.CompilerParams(use_tc_tiling_on_sc=False),
        scratch_types=[
            pltpu.VMEM((nchunk, _K), jnp.int32),
            pltpu.VMEM((_GROUP_ROWS, w), jnp.float32),
            pltpu.VMEM((_GROUP_ROWS, w), jnp.float32),
            pltpu.SemaphoreType.DMA,
            pltpu.SemaphoreType.DMA,
            pltpu.SemaphoreType.DMA,
            pltpu.SemaphoreType.DMA,
            pltpu.SemaphoreType.DMA,
        ],
    )
    def gather_rows(x_hbm, i_hbm, o_hbm, idx_v, buf_a, buf_b, isem, gs_a, gs_b, ss_a, ss_b):
        wid = lax.axis_index("subcore") * 2 + lax.axis_index("core")
        base = wid * rows_per_worker
        bufs = (buf_a, buf_b)
        gsems = (gs_a, gs_b)
        ssems = (ss_a, ss_b)

        pltpu.async_copy(i_hbm.at[wid], idx_v, isem).wait()

        @pl.loop(0, ngroup, step=2)
        def _(g0):
            for s in range(2):
                g = g0 + s

                @pl.when(g >= 2)
                def _():
                    pltpu.make_async_copy(
                        bufs[s],
                        o_hbm.at[pl.ds(base + (g - 2) * _GROUP_ROWS, _GROUP_ROWS)],
                        ssems[s],
                    ).wait()

                copies = [
                    pltpu.async_copy(
                        x_hbm.at[idx_v.at[g * _GC + j]],
                        bufs[s].at[pl.ds(j * _K, _K)],
                        gsems[s],
                    )
                    for j in range(_GC)
                ]
                for cp in copies:
                    cp.wait()
                pltpu.async_copy(
                    bufs[s],
                    o_hbm.at[pl.ds(base + g * _GROUP_ROWS, _GROUP_ROWS)],
                    ssems[s],
                )

        for s in range(2):
            g_last = ngroup - 2 + s
            pltpu.make_async_copy(
                bufs[s],
                o_hbm.at[pl.ds(base + g_last * _GROUP_ROWS, _GROUP_ROWS)],
                ssems[s],
            ).wait()

    out = gather_rows(rows, idx)
    return out.reshape(B, C, H, W)
